# Initial kernel scaffold; baseline (speedup 1.0000x reference)
import jax
import jax.numpy as jnp
from jax.experimental import pallas as pl


def kernel(pts, nn_idx, et_W1, et_b1, et_W2, et_b2, l0_Wmsg, l0_bmsg, l0_Wself, l0_bself, g_Wmsg, g_bmsg, g_Wself, g_bself):
    raise NotImplementedError("write your pallas kernel here")


# speedup vs baseline: 22.8683x; 22.8683x over previous
import functools

import jax
import jax.numpy as jnp
from jax import lax
from jax.experimental import pallas as pl
from jax.experimental.pallas import tpu as pltpu
from jax.experimental.pallas import tpu_sc as plsc

N = 10000
K = 16
C = 64
T = 4
ANUM = 2
E = N * K
H = 32

NC = 2
NS = 16
NW = NC * NS
EW = E // NW
CH = 40
NBUF = 5
NGRP = EW // (CH * NBUF)

NB = 1000
GRID = N // NB


def _sc_gather_body(table_hbm, idx_hbm, out_hbm, tab_sh, idx_v, bufs, gsem, osem):
    c = lax.axis_index("c")
    s = lax.axis_index("s")
    wid = s * NC + c

    @pl.when(s == 0)
    def _stage():
        pltpu.sync_copy(table_hbm, tab_sh)

    plsc.subcore_barrier()

    base = wid * EW
    pltpu.sync_copy(idx_hbm.at[pl.ds(base, EW)], idx_v)

    def group(g, carry):
        gh = []
        for b in range(NBUF):
            off = (g * NBUF + b) * CH
            gh.append(pltpu.async_copy(
                tab_sh.at[idx_v.at[pl.ds(off, CH)]], bufs.at[b], gsem))
        oh = []
        for b in range(NBUF):
            gh[b].wait()
            off = (g * NBUF + b) * CH
            oh.append(pltpu.async_copy(
                bufs.at[b], out_hbm.at[pl.ds(base + off, CH)], osem))
        for b in range(NBUF):
            oh[b].wait()
        return carry

    lax.fori_loop(0, NGRP, group, 0)


@functools.cache
def _sc_gather():
    return pl.kernel(
        _sc_gather_body,
        mesh=plsc.VectorSubcoreMesh(core_axis_name="c", subcore_axis_name="s"),
        out_type=jax.ShapeDtypeStruct((E, 128), jnp.float32),
        scratch_types=[
            pltpu.VMEM_SHARED((N, 128), jnp.float32),
            pltpu.VMEM((EW,), jnp.int32),
            pltpu.VMEM((NBUF, CH, 128), jnp.float32),
            pltpu.SemaphoreType.DMA,
            pltpu.SemaphoreType.DMA,
        ],
    )


def _edge_softmax(grows, tabblk, b1_ref, w2t_ref, b2_ref):
    p1g = grows.reshape(NB, K, 128)[:, :, C:C + H]
    p1c = tabblk[:, C:C + H]
    h = jnp.maximum(
        (p1g - p1c[:, None, :]).reshape(NB * K, H) + b1_ref[...], 0.0)
    lg = jnp.dot(h, w2t_ref[...], preferred_element_type=jnp.float32) + b2_ref[...]
    lg = lg - jnp.max(lg, axis=1, keepdims=True)
    ex = jnp.exp(lg)
    et = ex / jnp.sum(ex, axis=1, keepdims=True)
    return et.reshape(NB, K, T)


def _aggregate(et3, nbr):
    return jnp.concatenate(
        [jnp.sum(et3[:, :, t:t + 1] * nbr, axis=1) for t in range(T)], axis=1)


def _phase0_body(ptsT_ref, w1t_ref, tab_ref):
    cen = ptsT_ref[...]
    p1 = jnp.dot(cen, w1t_ref[...], preferred_element_type=jnp.float32)
    tab_ref[...] = jnp.concatenate(
        [cen, p1, jnp.zeros((NB, 128 - C - H), jnp.float32)], axis=1)


def _phase0_call(ptsT, w1t):
    return pl.pallas_call(
        _phase0_body,
        grid=(GRID,),
        in_specs=[
            pl.BlockSpec((NB, C), lambda i: (i, 0)),
            pl.BlockSpec((C, H), lambda i: (0, 0)),
        ],
        out_specs=pl.BlockSpec((NB, 128), lambda i: (i, 0)),
        out_shape=jax.ShapeDtypeStruct((N, 128), jnp.float32),
        compiler_params=pltpu.CompilerParams(
            dimension_semantics=("arbitrary",)),
    )(ptsT, w1t)


def _phase1_body(g0_ref, tab_ref, b1_ref, w2t_ref, b2_ref,
                 wmsgt_ref, wselft_ref, bias0_ref,
                 x1_ref, glb_ref):
    i = pl.program_id(0)
    g0 = g0_ref[...]
    tab = tab_ref[...]
    cen = tab[:, :C]
    et3 = _edge_softmax(g0, tab, b1_ref, w2t_ref, b2_ref)
    nbr = g0.reshape(NB, K, 128)[:, :, :C]
    msg = _aggregate(et3, nbr)
    out0 = (jnp.dot(msg, wmsgt_ref[...], preferred_element_type=jnp.float32)
            + jnp.dot(cen, wselft_ref[...], preferred_element_type=jnp.float32)
            + bias0_ref[...] + cen)
    x1 = jnp.maximum(out0, 0.0)
    x1_ref[...] = jnp.concatenate(
        [x1, tab[:, C:C + H], jnp.zeros((NB, 128 - C - H), jnp.float32)],
        axis=1)

    @pl.when(i == 0)
    def _init():
        glb_ref[...] = jnp.zeros((8, C), jnp.float32)

    mx = jnp.max(x1, axis=0, keepdims=True)
    glb_ref[...] = jnp.maximum(glb_ref[...], jnp.broadcast_to(mx, (8, C)))


def _phase1_call(g0, tab0, b1, w2t, b2, wmsgt, wselft, bias0):
    return pl.pallas_call(
        _phase1_body,
        grid=(GRID,),
        in_specs=[
            pl.BlockSpec((NB * K, 128), lambda i: (i, 0)),
            pl.BlockSpec((NB, 128), lambda i: (i, 0)),
            pl.BlockSpec((1, H), lambda i: (0, 0)),
            pl.BlockSpec((H, T), lambda i: (0, 0)),
            pl.BlockSpec((1, T), lambda i: (0, 0)),
            pl.BlockSpec((T * C, C), lambda i: (0, 0)),
            pl.BlockSpec((C, C), lambda i: (0, 0)),
            pl.BlockSpec((1, C), lambda i: (0, 0)),
        ],
        out_specs=[
            pl.BlockSpec((NB, 128), lambda i: (i, 0)),
            pl.BlockSpec((8, C), lambda i: (0, 0)),
        ],
        out_shape=[
            jax.ShapeDtypeStruct((N, 128), jnp.float32),
            jax.ShapeDtypeStruct((8, C), jnp.float32),
        ],
        compiler_params=pltpu.CompilerParams(
            dimension_semantics=("arbitrary",)),
    )(g0, tab0, b1, w2t, b2, wmsgt, wselft, bias0)


def _phase2_body(g1_ref, x1tab_ref, glb_ref, b1_ref, w2t_ref, b2_ref,
                 wmx_ref, m_ref, m2_ref, wsx_ref, biasg_ref, out_ref):
    glb = jnp.max(glb_ref[...], axis=0, keepdims=True)
    glb128 = jnp.concatenate([glb, glb], axis=1)
    wt = jnp.dot(glb128, m_ref[...], preferred_element_type=jnp.float32)
    v = jnp.dot(glb128, m2_ref[...], preferred_element_type=jnp.float32)
    g1 = g1_ref[...]
    x1tab = x1tab_ref[...]
    et3 = _edge_softmax(g1, x1tab, b1_ref, w2t_ref, b2_ref)
    s = jnp.sum(et3, axis=1)
    nbr = g1.reshape(NB, K, 128)[:, :, :C]
    msg = _aggregate(et3, nbr)
    out = (jnp.dot(msg, wmx_ref[...], preferred_element_type=jnp.float32)
           + jnp.dot(x1tab[:, :C], wsx_ref[...],
                     preferred_element_type=jnp.float32)
           + v + biasg_ref[...])
    for t in range(T):
        out = out + s[:, t:t + 1] * wt[:, t * C:(t + 1) * C]
    out_ref[...] = jnp.maximum(out, 0.0)


def _phase2_call(g1, x1tab, glb8, b1, w2t, b2, wmx, m, m2, wsx, biasg):
    return pl.pallas_call(
        _phase2_body,
        grid=(GRID,),
        in_specs=[
            pl.BlockSpec((NB * K, 128), lambda i: (i, 0)),
            pl.BlockSpec((NB, 128), lambda i: (i, 0)),
            pl.BlockSpec((8, C), lambda i: (0, 0)),
            pl.BlockSpec((1, H), lambda i: (0, 0)),
            pl.BlockSpec((H, T), lambda i: (0, 0)),
            pl.BlockSpec((1, T), lambda i: (0, 0)),
            pl.BlockSpec((T * C, C), lambda i: (0, 0)),
            pl.BlockSpec((128, T * C), lambda i: (0, 0)),
            pl.BlockSpec((128, C), lambda i: (0, 0)),
            pl.BlockSpec((C, C), lambda i: (0, 0)),
            pl.BlockSpec((1, C), lambda i: (0, 0)),
        ],
        out_specs=pl.BlockSpec((NB, C), lambda i: (i, 0)),
        out_shape=jax.ShapeDtypeStruct((N, C), jnp.float32),
        compiler_params=pltpu.CompilerParams(
            dimension_semantics=("arbitrary",)),
    )(g1, x1tab, glb8, b1, w2t, b2, wmx, m, m2, wsx, biasg)


def kernel(pts, nn_idx, et_W1, et_b1, et_W2, et_b2,
           l0_Wmsg, l0_bmsg, l0_Wself, l0_bself,
           g_Wmsg, g_bmsg, g_Wself, g_bself):
    ptsT = pts[0].T
    idx_flat = nn_idx[0].reshape(E)

    w1t = et_W1.T
    b1 = et_b1.reshape(1, H)
    b2 = et_b2.reshape(1, T)
    w2t = et_W2.T
    wmsgt = l0_Wmsg.T
    wselft = l0_Wself.T
    bias0 = (l0_bmsg + l0_bself).reshape(1, C)
    wg3 = g_Wmsg.reshape(C, T, C * (ANUM + 1))
    wmx = wg3[:, :, 128:].transpose(1, 2, 0).reshape(T * C, C)
    m = wg3[:, :, :128].transpose(2, 1, 0).reshape(128, T * C)
    m2 = g_Wself[:, :128].T
    wsx = g_Wself[:, 128:].T
    biasg = (g_bmsg + g_bself).reshape(1, C)

    gather = _sc_gather()
    tab0 = _phase0_call(ptsT, w1t)
    g0 = gather(tab0, idx_flat)
    x1tab, glb8 = _phase1_call(g0, tab0, b1, w2t, b2, wmsgt, wselft, bias0)
    g1 = gather(x1tab, idx_flat)
    out2 = _phase2_call(g1, x1tab, glb8, b1, w2t, b2,
                        wmx, m, m2, wsx, biasg)
    return jnp.broadcast_to(out2.T[None, None, :, :, None], (1, ANUM, C, N, 1))


# speedup vs baseline: 23.7408x; 1.0382x over previous
import functools

import jax
import jax.numpy as jnp
from jax import lax
from jax.experimental import pallas as pl
from jax.experimental.pallas import tpu as pltpu
from jax.experimental.pallas import tpu_sc as plsc

N = 10000
K = 16
C = 64
T = 4
ANUM = 2
E = N * K
H = 32

NC = 2
NS = 16
NW = NC * NS
EW = E // NW
CH = 40
NBUF = 5
NGRP = EW // (CH * NBUF)

NB = 1000
GRID = N // NB


def _sc_gather_body(table_hbm, idx_hbm, out_hbm, tab_sh, idx_v, bufs, gsem, osem):
    c = lax.axis_index("c")
    s = lax.axis_index("s")
    wid = s * NC + c

    @pl.when(s == 0)
    def _stage():
        pltpu.sync_copy(table_hbm, tab_sh)

    plsc.subcore_barrier()

    base = wid * EW
    pltpu.sync_copy(idx_hbm.at[pl.ds(base, EW)], idx_v)

    def group(g, carry):
        gh = []
        for b in range(NBUF):
            off = (g * NBUF + b) * CH
            gh.append(pltpu.async_copy(
                tab_sh.at[idx_v.at[pl.ds(off, CH)]], bufs.at[b], gsem))
        oh = []
        for b in range(NBUF):
            gh[b].wait()
            off = (g * NBUF + b) * CH
            oh.append(pltpu.async_copy(
                bufs.at[b], out_hbm.at[pl.ds(base + off, CH)], osem))
        for b in range(NBUF):
            oh[b].wait()
        return carry

    lax.fori_loop(0, NGRP, group, 0)


@functools.cache
def _sc_gather():
    return pl.kernel(
        _sc_gather_body,
        mesh=plsc.VectorSubcoreMesh(core_axis_name="c", subcore_axis_name="s"),
        out_type=jax.ShapeDtypeStruct((E, 128), jnp.float32),
        scratch_types=[
            pltpu.VMEM_SHARED((N, 128), jnp.float32),
            pltpu.VMEM((EW,), jnp.int32),
            pltpu.VMEM((NBUF, CH, 128), jnp.float32),
            pltpu.SemaphoreType.DMA,
            pltpu.SemaphoreType.DMA,
        ],
    )


NGB = NB * K // 128


def _edge_weights(grows, tabblk, b1_ref, w2t_ref, b2_ref):
    p1g = grows.reshape(NB, K, 128)[:, :, C:C + H]
    p1c = tabblk[:, C:C + H]
    h = jnp.maximum(
        (p1g - p1c[:, None, :]).reshape(NB * K, H) + b1_ref[...], 0.0)
    lg = jnp.dot(h, w2t_ref[...], preferred_element_type=jnp.float32) + b2_ref[...]
    lg = lg - jnp.max(lg, axis=1, keepdims=True)
    ex = jnp.exp(lg)
    et = ex / jnp.sum(ex, axis=1, keepdims=True)
    lane8 = lax.broadcasted_iota(jnp.int32, (NB * K, 8), 1)
    row8 = lax.broadcasted_iota(jnp.int32, (NB * K, 8), 0)
    m8 = lane8 == (row8 // K) % 8
    return jnp.concatenate(
        [jnp.where(m8, et[:, t:t + 1], 0.0) for t in range(T)], axis=1)


def _masked_msg(En, grows):
    En3 = En.reshape(NGB, 128, T * 8)
    R3 = grows.reshape(NGB, 128, 128)
    M = lax.dot_general(En3, R3, (((1,), (1,)), ((0,), (0,))),
                        preferred_element_type=jnp.float32)
    return M.reshape(NGB, T, 8, 128)


def _phase0_body(ptsT_ref, w1t_ref, tab_ref):
    cen = ptsT_ref[...]
    p1 = jnp.dot(cen, w1t_ref[...], preferred_element_type=jnp.float32)
    tab_ref[...] = jnp.concatenate(
        [cen, p1, jnp.zeros((NB, 128 - C - H), jnp.float32)], axis=1)


def _phase0_call(ptsT, w1t):
    return pl.pallas_call(
        _phase0_body,
        grid=(GRID,),
        in_specs=[
            pl.BlockSpec((NB, C), lambda i: (i, 0)),
            pl.BlockSpec((C, H), lambda i: (0, 0)),
        ],
        out_specs=pl.BlockSpec((NB, 128), lambda i: (i, 0)),
        out_shape=jax.ShapeDtypeStruct((N, 128), jnp.float32),
        compiler_params=pltpu.CompilerParams(
            dimension_semantics=("arbitrary",)),
    )(ptsT, w1t)


def _phase1_body(g0_ref, tab_ref, b1_ref, w2t_ref, b2_ref,
                 w4_ref, wselft_ref, bias0_ref,
                 x1_ref, glb_ref):
    i = pl.program_id(0)
    g0 = g0_ref[...]
    tab = tab_ref[...]
    cen = tab[:, :C]
    En = _edge_weights(g0, tab, b1_ref, w2t_ref, b2_ref)
    M4 = _masked_msg(En, g0)
    out0 = (jnp.dot(cen, wselft_ref[...], preferred_element_type=jnp.float32)
            + bias0_ref[...] + cen)
    for t in range(T):
        Mt = M4[:, t].reshape(NB, 128)
        out0 = out0 + jnp.dot(Mt, w4_ref[t],
                              preferred_element_type=jnp.float32)
    x1 = jnp.maximum(out0, 0.0)
    x1_ref[...] = jnp.concatenate(
        [x1, tab[:, C:C + H], jnp.ones((NB, 1), jnp.float32),
         jnp.zeros((NB, 128 - C - H - 1), jnp.float32)],
        axis=1)

    @pl.when(i == 0)
    def _init():
        glb_ref[...] = jnp.zeros((8, C), jnp.float32)

    mx = jnp.max(x1, axis=0, keepdims=True)
    glb_ref[...] = jnp.maximum(glb_ref[...], jnp.broadcast_to(mx, (8, C)))


def _phase1_call(g0, tab0, b1, w2t, b2, w4, wselft, bias0):
    return pl.pallas_call(
        _phase1_body,
        grid=(GRID,),
        in_specs=[
            pl.BlockSpec((NB * K, 128), lambda i: (i, 0)),
            pl.BlockSpec((NB, 128), lambda i: (i, 0)),
            pl.BlockSpec((1, H), lambda i: (0, 0)),
            pl.BlockSpec((H, T), lambda i: (0, 0)),
            pl.BlockSpec((1, T), lambda i: (0, 0)),
            pl.BlockSpec((T, 128, C), lambda i: (0, 0, 0)),
            pl.BlockSpec((C, C), lambda i: (0, 0)),
            pl.BlockSpec((1, C), lambda i: (0, 0)),
        ],
        out_specs=[
            pl.BlockSpec((NB, 128), lambda i: (i, 0)),
            pl.BlockSpec((8, C), lambda i: (0, 0)),
        ],
        out_shape=[
            jax.ShapeDtypeStruct((N, 128), jnp.float32),
            jax.ShapeDtypeStruct((8, C), jnp.float32),
        ],
        compiler_params=pltpu.CompilerParams(
            dimension_semantics=("arbitrary",)),
    )(g0, tab0, b1, w2t, b2, w4, wselft, bias0)


def _phase2_body(g1_ref, x1tab_ref, glb_ref, b1_ref, w2t_ref, b2_ref,
                 w4g_ref, m_ref, m2_ref, wsx_ref, biasg_ref, out_ref):
    glb = jnp.max(glb_ref[...], axis=0, keepdims=True)
    glb128 = jnp.concatenate([glb, glb], axis=1)
    wt = jnp.dot(glb128, m_ref[...], preferred_element_type=jnp.float32)
    v = jnp.dot(glb128, m2_ref[...], preferred_element_type=jnp.float32)
    g1 = g1_ref[...]
    x1tab = x1tab_ref[...]
    En = _edge_weights(g1, x1tab, b1_ref, w2t_ref, b2_ref)
    M4 = _masked_msg(En, g1)
    out = (jnp.dot(x1tab[:, :C], wsx_ref[...],
                   preferred_element_type=jnp.float32)
           + v + biasg_ref[...])
    for t in range(T):
        Mt = M4[:, t].reshape(NB, 128)
        out = (out + jnp.dot(Mt, w4g_ref[t],
                             preferred_element_type=jnp.float32)
               + Mt[:, C + H:C + H + 1] * wt[:, t * C:(t + 1) * C])
    out_ref[...] = jnp.maximum(out, 0.0)


def _phase2_call(g1, x1tab, glb8, b1, w2t, b2, w4g, m, m2, wsx, biasg):
    return pl.pallas_call(
        _phase2_body,
        grid=(GRID,),
        in_specs=[
            pl.BlockSpec((NB * K, 128), lambda i: (i, 0)),
            pl.BlockSpec((NB, 128), lambda i: (i, 0)),
            pl.BlockSpec((8, C), lambda i: (0, 0)),
            pl.BlockSpec((1, H), lambda i: (0, 0)),
            pl.BlockSpec((H, T), lambda i: (0, 0)),
            pl.BlockSpec((1, T), lambda i: (0, 0)),
            pl.BlockSpec((T, 128, C), lambda i: (0, 0, 0)),
            pl.BlockSpec((128, T * C), lambda i: (0, 0)),
            pl.BlockSpec((128, C), lambda i: (0, 0)),
            pl.BlockSpec((C, C), lambda i: (0, 0)),
            pl.BlockSpec((1, C), lambda i: (0, 0)),
        ],
        out_specs=pl.BlockSpec((NB, C), lambda i: (i, 0)),
        out_shape=jax.ShapeDtypeStruct((N, C), jnp.float32),
        compiler_params=pltpu.CompilerParams(
            dimension_semantics=("arbitrary",)),
    )(g1, x1tab, glb8, b1, w2t, b2, w4g, m, m2, wsx, biasg)


def kernel(pts, nn_idx, et_W1, et_b1, et_W2, et_b2,
           l0_Wmsg, l0_bmsg, l0_Wself, l0_bself,
           g_Wmsg, g_bmsg, g_Wself, g_bself):
    ptsT = pts[0].T
    idx_flat = nn_idx[0].reshape(E)

    w1t = et_W1.T
    b1 = et_b1.reshape(1, H)
    b2 = et_b2.reshape(1, T)
    w2t = et_W2.T
    pad = jnp.zeros((T, 128 - C, C), jnp.float32)
    w4 = jnp.concatenate([l0_Wmsg.T.reshape(T, C, C), pad], axis=1)
    wselft = l0_Wself.T
    bias0 = (l0_bmsg + l0_bself).reshape(1, C)
    wg3 = g_Wmsg.reshape(C, T, C * (ANUM + 1))
    wmx = wg3[:, :, 128:].transpose(1, 2, 0).reshape(T * C, C)
    w4g = jnp.concatenate([wmx.reshape(T, C, C), pad], axis=1)
    m = wg3[:, :, :128].transpose(2, 1, 0).reshape(128, T * C)
    m2 = g_Wself[:, :128].T
    wsx = g_Wself[:, 128:].T
    biasg = (g_bmsg + g_bself).reshape(1, C)

    gather = _sc_gather()
    tab0 = _phase0_call(ptsT, w1t)
    g0 = gather(tab0, idx_flat)
    x1tab, glb8 = _phase1_call(g0, tab0, b1, w2t, b2, w4, wselft, bias0)
    g1 = gather(x1tab, idx_flat)
    out2 = _phase2_call(g1, x1tab, glb8, b1, w2t, b2,
                        w4g, m, m2, wsx, biasg)
    return jnp.broadcast_to(out2.T[None, None, :, :, None], (1, ANUM, C, N, 1))
